# baseline (speedup 1.0000x reference)
import jax
import jax.numpy as jnp
from jax import lax
from jax.experimental import pallas as pl
from jax.experimental.pallas import tpu as pltpu
from jax.experimental.pallas import tpu_sc as plsc

N = 10000
E = 160000
D_IN = 256
HID = 64
HEADS = 8
D_OUT = 64
F1 = HEADS * HID

NC = 2
NS = 16
SB = 250
NB1 = E // NS // SB
NB2 = E // (NC * NS) // SB
NP = 10112
RPT = NP // NS

BN = 1000
NBLK = N // BN
CW = 64
NCHUNK = F1 // CW
CPC = NCHUNK // NC

_HIGH = lax.Precision.HIGHEST
_SC_PARAMS = pltpu.CompilerParams(use_tc_tiling_on_sc=False)


def _stage_a_body(x_ref, w_ref, aa_ref, h4_ref, aaout_ref):
    c = pl.program_id(1)
    h = jnp.dot(x_ref[...], w_ref[0], preferred_element_type=jnp.float32)
    h4_ref[0] = h
    part = jnp.dot(h, aa_ref[0], preferred_element_type=jnp.float32)

    @pl.when(c == 0)
    def _():
        aaout_ref[...] = part

    @pl.when(c > 0)
    def _():
        aaout_ref[...] = aaout_ref[...] + part


def _stage_a(x, W1r, AA1r):
    return pl.pallas_call(
        _stage_a_body,
        grid=(NBLK, NCHUNK),
        in_specs=[
            pl.BlockSpec((BN, D_IN), lambda i, c: (i, 0)),
            pl.BlockSpec((1, D_IN, CW), lambda i, c: (c, 0, 0)),
            pl.BlockSpec((1, CW, 32), lambda i, c: (c, 0, 0)),
        ],
        out_specs=[
            pl.BlockSpec((1, BN, CW), lambda i, c: (c, i, 0)),
            pl.BlockSpec((BN, 32), lambda i, c: (i, 0)),
        ],
        out_shape=[
            jax.ShapeDtypeStruct((NCHUNK, N, CW), jnp.float32),
            jax.ShapeDtypeStruct((N, 32), jnp.float32),
        ],
    )(x, W1r, AA1r)


G = 2


_DYN_GATHER_DNUMS = lax.GatherDimensionNumbers(
    offset_dims=(), collapsed_slice_dims=(0,), start_index_map=(0,))


def _splat_lane(row, lane_vec):
    return lax.gather(row, lane_vec[:, None], _DYN_GATHER_DNUMS,
                      slice_sizes=(1,),
                      mode=lax.GatherScatterMode.PROMISE_IN_BOUNDS)


_UNROLL = 5


def _fused_weight_loop(expb_q, hrow_q, lane_vec):
    def w_body(k5, c2):
        for u in range(_UNROLL):
            k = k5 * _UNROLL + u
            w = _splat_lane(expb_q[k], lane_vec)
            for j in range(CW // 16):
                hrow_q[k, pl.ds(j * 16, 16)] = (
                    hrow_q[k, pl.ds(j * 16, 16)] * w)
        return c2

    lax.fori_loop(0, SB // _UNROLL, w_body, 0)


def _exp_loop(abuf_q, bbuf_q, expb_q):
    def edge_body(k5, c2):
        for u in range(_UNROLL):
            k = k5 * _UNROLL + u
            e = abuf_q[k] + bbuf_q[k]
            e = jnp.where(e >= 0.0, e, 0.2 * e)
            expb_q[k] = jnp.exp(e)
        return c2

    lax.fori_loop(0, SB // _UNROLL, edge_body, 0)


def _stage_b_body(h4, asd, add_, srcr, dstr, z64, z16,
                  num1, den1, ecache,
                  idx_s, idx_d, abufs, bbufs, expbs, hrows,
                  num_acc, den1_acc, *sems):
    semw = sems[2 * G]
    cid = lax.axis_index("c")
    sid = lax.axis_index("s")
    rows0 = sid * RPT

    pltpu.sync_copy(srcr.at[sid], idx_s)
    pltpu.sync_copy(dstr.at[sid], idx_d)
    pltpu.sync_copy(z16, den1_acc.at[pl.ds(rows0, RPT)])

    for t in range(CPC):
        chunk = cid * CPC + t
        lane_vec = jnp.full((16,), chunk, jnp.int32)
        pltpu.sync_copy(z64, num_acc.at[pl.ds(rows0, RPT)])
        plsc.subcore_barrier()

        def group_body(g, carry, t=t, chunk=chunk, lane_vec=lane_vec):
            b0 = g * G
            wprev = [None]

            def wissue(desc_fn):
                if wprev[0] is not None:
                    wprev[0].wait()
                wprev[0] = desc_fn()
            adescs, hdescs = [], []
            for q in range(G):
                b = b0 + q
                if t == 0:
                    adescs.append((
                        pltpu.async_copy(asd.at[idx_s.at[b]],
                                         abufs.at[q], sems[q]),
                        pltpu.async_copy(add_.at[idx_d.at[b]],
                                         bbufs.at[q], sems[q]),
                    ))
                else:
                    adescs.append((
                        pltpu.async_copy(ecache.at[cid].at[sid].at[b],
                                         expbs.at[q], sems[q]),
                    ))
                hdescs.append(pltpu.async_copy(
                    h4.at[chunk].at[idx_s.at[b]], hrows.at[q],
                    sems[G + q]))
            for q in range(G):
                b = b0 + q
                for d in adescs[q]:
                    d.wait()
                if t == 0:
                    _exp_loop(abufs.at[q], bbufs.at[q], expbs.at[q])
                    wissue(lambda q=q, b=b: pltpu.async_copy(
                        expbs.at[q], ecache.at[cid].at[sid].at[b], semw))
                    wissue(lambda q=q, b=b: pltpu.async_copy(
                        expbs.at[q], den1_acc.at[idx_d.at[b]], semw,
                        add=True))
                hdescs[q].wait()
                _fused_weight_loop(expbs.at[q], hrows.at[q], lane_vec)
                wissue(lambda q=q, b=b: pltpu.async_copy(
                    hrows.at[q], num_acc.at[idx_d.at[b]], semw, add=True))
            wprev[0].wait()
            return carry

        lax.fori_loop(0, NB1 // G, group_body, 0)
        plsc.subcore_barrier()
        pltpu.sync_copy(num_acc.at[pl.ds(rows0, RPT)],
                        num1.at[chunk].at[pl.ds(rows0, RPT)])
        plsc.subcore_barrier()

    pltpu.sync_copy(den1_acc.at[pl.ds(rows0, RPT)],
                    den1.at[cid].at[pl.ds(rows0, RPT)])


def _stage_b(h4, asdup, addup, srcr, dstr, z64, z16):
    return pl.kernel(
        _stage_b_body,
        out_type=[
            jax.ShapeDtypeStruct((NCHUNK, NP, CW), jnp.float32),
            jax.ShapeDtypeStruct((NC, NP, 16), jnp.float32),
            jax.ShapeDtypeStruct((NC, NS, NB1, SB, 16), jnp.float32),
        ],
        mesh=plsc.VectorSubcoreMesh(core_axis_name="c", subcore_axis_name="s"),
        compiler_params=_SC_PARAMS,
        scratch_types=[
            pltpu.VMEM((NB1, SB), jnp.int32),
            pltpu.VMEM((NB1, SB), jnp.int32),
            pltpu.VMEM((G, SB, 16), jnp.float32),
            pltpu.VMEM((G, SB, 16), jnp.float32),
            pltpu.VMEM((G, SB, 16), jnp.float32),
            pltpu.VMEM((G, SB, CW), jnp.float32),
            pltpu.VMEM_SHARED((NP, CW), jnp.float32),
            pltpu.VMEM_SHARED((NP, 16), jnp.float32),
        ] + [pltpu.SemaphoreType.DMA] * (2 * G + 1),
    )(h4, asdup, addup, srcr, dstr, z64, z16)


def _stage_c_body(num_ref, den_ref, b1_ref, w2_ref, aa2_ref, exp8_ref,
                  h2_ref, aaout_ref):
    hcat = jnp.concatenate([num_ref[i] for i in range(NCHUNK)], axis=-1)
    den = den_ref[0][:, 0:HEADS]
    recip = 1.0 / (den + 1e-16)
    denex = jnp.dot(recip, exp8_ref[...], preferred_element_type=jnp.float32)
    out1 = hcat * denex + b1_ref[...]
    t = jnp.where(out1 > 0.0, out1, jnp.exp(out1) - 1.0)
    h2 = jnp.dot(t, w2_ref[...], preferred_element_type=jnp.float32)
    h2_ref[...] = h2
    aaout_ref[...] = jnp.dot(h2, aa2_ref[...],
                             preferred_element_type=jnp.float32)


def _stage_c(num1, den1, b1, W2, AA2, EXP8):
    return pl.pallas_call(
        _stage_c_body,
        grid=(NBLK,),
        in_specs=[
            pl.BlockSpec((NCHUNK, BN, CW), lambda i: (0, i, 0)),
            pl.BlockSpec((1, BN, 16), lambda i: (0, i, 0)),
            pl.BlockSpec((1, F1), lambda i: (0, 0)),
            pl.BlockSpec((F1, D_OUT), lambda i: (0, 0)),
            pl.BlockSpec((D_OUT, 32), lambda i: (0, 0)),
            pl.BlockSpec((HEADS, F1), lambda i: (0, 0)),
        ],
        out_specs=[
            pl.BlockSpec((BN, D_OUT), lambda i: (i, 0)),
            pl.BlockSpec((BN, 32), lambda i: (i, 0)),
        ],
        out_shape=[
            jax.ShapeDtypeStruct((N, D_OUT), jnp.float32),
            jax.ShapeDtypeStruct((N, 32), jnp.float32),
        ],
    )(num1, den1, b1, W2, AA2, EXP8)


def _stage_d_body(h2, asd, add_, srcr, dstr, z64, z16,
                  num2, den2,
                  idx_s, idx_d, abufs, bbufs, expbs, hrows,
                  num2_acc, den2_acc, *sems):
    semw = sems[2 * G]
    cid = lax.axis_index("c")
    sid = lax.axis_index("s")
    wid = sid * NC + cid
    rows0 = sid * RPT

    pltpu.sync_copy(srcr.at[wid], idx_s)
    pltpu.sync_copy(dstr.at[wid], idx_d)
    pltpu.sync_copy(z64, num2_acc.at[pl.ds(rows0, RPT)])
    pltpu.sync_copy(z16, den2_acc.at[pl.ds(rows0, RPT)])
    plsc.subcore_barrier()

    def group_body(g, carry):
        b0 = g * G
        wprev = [None]

        def wissue(desc_fn):
            if wprev[0] is not None:
                wprev[0].wait()
            wprev[0] = desc_fn()

        adescs, hdescs = [], []
        for q in range(G):
            b = b0 + q
            adescs.append((
                pltpu.async_copy(asd.at[idx_s.at[b]], abufs.at[q],
                                 sems[q]),
                pltpu.async_copy(add_.at[idx_d.at[b]], bbufs.at[q],
                                 sems[q]),
            ))
            hdescs.append(pltpu.async_copy(
                h2.at[idx_s.at[b]], hrows.at[q], sems[G + q]))
        for q in range(G):
            b = b0 + q
            for d in adescs[q]:
                d.wait()
            _exp_loop(abufs.at[q], bbufs.at[q], expbs.at[q])
            wissue(lambda q=q, b=b: pltpu.async_copy(
                expbs.at[q], den2_acc.at[idx_d.at[b]], semw, add=True))
            hdescs[q].wait()
            _fused_weight_loop(expbs.at[q], hrows.at[q],
                               jnp.zeros((16,), jnp.int32))
            wissue(lambda q=q, b=b: pltpu.async_copy(
                hrows.at[q], num2_acc.at[idx_d.at[b]], semw, add=True))
        wprev[0].wait()
        return carry

    lax.fori_loop(0, NB2 // G, group_body, 0)
    plsc.subcore_barrier()
    pltpu.sync_copy(num2_acc.at[pl.ds(rows0, RPT)],
                    num2.at[cid].at[pl.ds(rows0, RPT)])
    pltpu.sync_copy(den2_acc.at[pl.ds(rows0, RPT)],
                    den2.at[cid].at[pl.ds(rows0, RPT)])


def _stage_d(h2, as2dup, ad2dup, srcr, dstr, z64, z16):
    return pl.kernel(
        _stage_d_body,
        out_type=[
            jax.ShapeDtypeStruct((NC, NP, D_OUT), jnp.float32),
            jax.ShapeDtypeStruct((NC, NP, 16), jnp.float32),
        ],
        mesh=plsc.VectorSubcoreMesh(core_axis_name="c", subcore_axis_name="s"),
        compiler_params=_SC_PARAMS,
        scratch_types=[
            pltpu.VMEM((NB2, SB), jnp.int32),
            pltpu.VMEM((NB2, SB), jnp.int32),
            pltpu.VMEM((G, SB, 16), jnp.float32),
            pltpu.VMEM((G, SB, 16), jnp.float32),
            pltpu.VMEM((G, SB, 16), jnp.float32),
            pltpu.VMEM((G, SB, D_OUT), jnp.float32),
            pltpu.VMEM_SHARED((NP, D_OUT), jnp.float32),
            pltpu.VMEM_SHARED((NP, 16), jnp.float32),
        ] + [pltpu.SemaphoreType.DMA] * (2 * G + 1),
    )(h2, as2dup, ad2dup, srcr, dstr, z64, z16)


def _stage_e_body(num2_ref, den2_ref, b2_ref, out_ref):
    s = num2_ref[0] + num2_ref[1]
    d = den2_ref[0][:, 0:1] + den2_ref[1][:, 0:1]
    out_ref[...] = s / (d + 1e-16) + b2_ref[...]


def _stage_e(num2, den2, b2):
    return pl.pallas_call(
        _stage_e_body,
        grid=(NBLK,),
        in_specs=[
            pl.BlockSpec((NC, BN, D_OUT), lambda i: (0, i, 0)),
            pl.BlockSpec((NC, BN, 16), lambda i: (0, i, 0)),
            pl.BlockSpec((1, D_OUT), lambda i: (0, 0)),
        ],
        out_specs=pl.BlockSpec((BN, D_OUT), lambda i: (i, 0)),
        out_shape=jax.ShapeDtypeStruct((N, D_OUT), jnp.float32),
    )(num2, den2, b2)


def kernel(x, edge_index, W1, a_src1, a_dst1, b1, W2, a_src2, a_dst2, b2):
    src = edge_index[0]
    dst = edge_index[1]

    eye8 = jnp.eye(HEADS, dtype=jnp.float32)
    blk_s = (a_src1[:, :, None] * eye8[:, None, :]).reshape(F1, HEADS)
    blk_d = (a_dst1[:, :, None] * eye8[:, None, :]).reshape(F1, HEADS)
    AA1 = jnp.concatenate([blk_s, blk_s, blk_d, blk_d], axis=1)
    AA1r = AA1.reshape(NCHUNK, CW, 32)
    W1r = W1.reshape(D_IN, NCHUNK, CW).transpose(1, 0, 2)

    AA2 = jnp.concatenate([
        jnp.broadcast_to(a_src2.reshape(D_OUT, 1), (D_OUT, 16)),
        jnp.broadcast_to(a_dst2.reshape(D_OUT, 1), (D_OUT, 16)),
    ], axis=1)
    EXP8 = (eye8[:, :, None] * jnp.ones((1, 1, HID), jnp.float32)
            ).reshape(HEADS, F1)

    h4, aa = _stage_a(x, W1r, AA1r)
    asdup = aa[:, :16]
    addup = aa[:, 16:]

    srcr1 = src.reshape(NS, NB1, SB)
    dstr1 = dst.reshape(NS, NB1, SB)
    z64 = jnp.zeros((RPT, CW), jnp.float32)
    z8 = jnp.zeros((RPT, 16), jnp.float32)

    num1, den1, _ = _stage_b(h4, asdup, addup, srcr1, dstr1, z64, z8)
    h2, aa2o = _stage_c(num1, den1, b1.reshape(1, F1), W2, AA2, EXP8)

    srcr2 = src.reshape(NC * NS, NB2, SB)
    dstr2 = dst.reshape(NC * NS, NB2, SB)
    num2, den2 = _stage_d(h2, aa2o[:, :16], aa2o[:, 16:], srcr2, dstr2,
                          z64, z8)
    return _stage_e(num2, den2, b2.reshape(1, D_OUT))


# speedup vs baseline: 1.0509x; 1.0509x over previous
import jax
import jax.numpy as jnp
from jax import lax
from jax.experimental import pallas as pl
from jax.experimental.pallas import tpu as pltpu
from jax.experimental.pallas import tpu_sc as plsc

N = 10000
E = 160000
D_IN = 256
HID = 64
HEADS = 8
D_OUT = 64
F1 = HEADS * HID

NC = 2
NS = 16
SB = 125
NB1 = E // NS // SB
NB2 = E // (NC * NS) // SB
NP = 10112
RPT = NP // NS

BN = 1000
NBLK = N // BN
CW = 64
NCHUNK = F1 // CW
CPC = NCHUNK // NC

_HIGH = lax.Precision.HIGHEST
_SC_PARAMS = pltpu.CompilerParams(use_tc_tiling_on_sc=False)


def _stage_a_body(x_ref, w_ref, aa_ref, h4_ref, aaout_ref):
    c = pl.program_id(1)
    h = jnp.dot(x_ref[...], w_ref[0], preferred_element_type=jnp.float32)
    h4_ref[0] = h
    part = jnp.dot(h, aa_ref[0], preferred_element_type=jnp.float32)

    @pl.when(c == 0)
    def _():
        aaout_ref[...] = part

    @pl.when(c > 0)
    def _():
        aaout_ref[...] = aaout_ref[...] + part


def _stage_a(x, W1r, AA1r):
    return pl.pallas_call(
        _stage_a_body,
        grid=(NBLK, NCHUNK),
        in_specs=[
            pl.BlockSpec((BN, D_IN), lambda i, c: (i, 0)),
            pl.BlockSpec((1, D_IN, CW), lambda i, c: (c, 0, 0)),
            pl.BlockSpec((1, CW, 32), lambda i, c: (c, 0, 0)),
        ],
        out_specs=[
            pl.BlockSpec((1, BN, CW), lambda i, c: (c, i, 0)),
            pl.BlockSpec((BN, 32), lambda i, c: (i, 0)),
        ],
        out_shape=[
            jax.ShapeDtypeStruct((NCHUNK, N, CW), jnp.float32),
            jax.ShapeDtypeStruct((N, 32), jnp.float32),
        ],
    )(x, W1r, AA1r)


G = 4


_DYN_GATHER_DNUMS = lax.GatherDimensionNumbers(
    offset_dims=(), collapsed_slice_dims=(0,), start_index_map=(0,))


def _splat_lane(row, lane_vec):
    return lax.gather(row, lane_vec[:, None], _DYN_GATHER_DNUMS,
                      slice_sizes=(1,),
                      mode=lax.GatherScatterMode.PROMISE_IN_BOUNDS)


_UNROLL = 5


def _fused_weight_loop(expb_q, hrow_q, lane_vec):
    def w_body(k5, c2):
        for u in range(_UNROLL):
            k = k5 * _UNROLL + u
            w = _splat_lane(expb_q[k], lane_vec)
            for j in range(CW // 16):
                hrow_q[k, pl.ds(j * 16, 16)] = (
                    hrow_q[k, pl.ds(j * 16, 16)] * w)
        return c2

    lax.fori_loop(0, SB // _UNROLL, w_body, 0)


def _exp_loop(abuf_q, bbuf_q, expb_q):
    def edge_body(k5, c2):
        for u in range(_UNROLL):
            k = k5 * _UNROLL + u
            e = abuf_q[k] + bbuf_q[k]
            e = jnp.where(e >= 0.0, e, 0.2 * e)
            expb_q[k] = jnp.exp(e)
        return c2

    lax.fori_loop(0, SB // _UNROLL, edge_body, 0)


def _stage_b_body(h4, asd, add_, srcr, dstr, z64, z16,
                  num1, den1, ecache,
                  idx_s, idx_d, abufs, bbufs, expbs, hrows,
                  num_acc, den1_acc, *sems):
    semw = sems[2 * G]
    cid = lax.axis_index("c")
    sid = lax.axis_index("s")
    rows0 = sid * RPT

    pltpu.sync_copy(srcr.at[sid], idx_s)
    pltpu.sync_copy(dstr.at[sid], idx_d)
    pltpu.sync_copy(z16, den1_acc.at[pl.ds(rows0, RPT)])

    for t in range(CPC):
        chunk = cid * CPC + t
        lane_vec = jnp.full((16,), chunk, jnp.int32)
        pltpu.sync_copy(z64, num_acc.at[pl.ds(rows0, RPT)])
        plsc.subcore_barrier()

        def group_body(g, carry, t=t, chunk=chunk, lane_vec=lane_vec):
            b0 = g * G
            wprev = [None]

            def wissue(desc_fn):
                if wprev[0] is not None:
                    wprev[0].wait()
                wprev[0] = desc_fn()
            adescs, hdescs = [], []
            for q in range(G):
                b = b0 + q
                if t == 0:
                    adescs.append((
                        pltpu.async_copy(asd.at[idx_s.at[b]],
                                         abufs.at[q], sems[q]),
                        pltpu.async_copy(add_.at[idx_d.at[b]],
                                         bbufs.at[q], sems[q]),
                    ))
                else:
                    adescs.append((
                        pltpu.async_copy(ecache.at[cid].at[sid].at[b],
                                         expbs.at[q], sems[q]),
                    ))
                hdescs.append(pltpu.async_copy(
                    h4.at[chunk].at[idx_s.at[b]], hrows.at[q],
                    sems[G + q]))
            for q in range(G):
                b = b0 + q
                for d in adescs[q]:
                    d.wait()
                if t == 0:
                    _exp_loop(abufs.at[q], bbufs.at[q], expbs.at[q])
                    wissue(lambda q=q, b=b: pltpu.async_copy(
                        expbs.at[q], ecache.at[cid].at[sid].at[b], semw))
                    wissue(lambda q=q, b=b: pltpu.async_copy(
                        expbs.at[q], den1_acc.at[idx_d.at[b]], semw,
                        add=True))
                hdescs[q].wait()
                _fused_weight_loop(expbs.at[q], hrows.at[q], lane_vec)
                wissue(lambda q=q, b=b: pltpu.async_copy(
                    hrows.at[q], num_acc.at[idx_d.at[b]], semw, add=True))
            wprev[0].wait()
            return carry

        lax.fori_loop(0, NB1 // G, group_body, 0)
        plsc.subcore_barrier()
        pltpu.sync_copy(num_acc.at[pl.ds(rows0, RPT)],
                        num1.at[chunk].at[pl.ds(rows0, RPT)])
        plsc.subcore_barrier()

    pltpu.sync_copy(den1_acc.at[pl.ds(rows0, RPT)],
                    den1.at[cid].at[pl.ds(rows0, RPT)])


def _stage_b(h4, asdup, addup, srcr, dstr, z64, z16):
    return pl.kernel(
        _stage_b_body,
        out_type=[
            jax.ShapeDtypeStruct((NCHUNK, NP, CW), jnp.float32),
            jax.ShapeDtypeStruct((NC, NP, 16), jnp.float32),
            jax.ShapeDtypeStruct((NC, NS, NB1, SB, 16), jnp.float32),
        ],
        mesh=plsc.VectorSubcoreMesh(core_axis_name="c", subcore_axis_name="s"),
        compiler_params=_SC_PARAMS,
        scratch_types=[
            pltpu.VMEM((NB1, SB), jnp.int32),
            pltpu.VMEM((NB1, SB), jnp.int32),
            pltpu.VMEM((G, SB, 16), jnp.float32),
            pltpu.VMEM((G, SB, 16), jnp.float32),
            pltpu.VMEM((G, SB, 16), jnp.float32),
            pltpu.VMEM((G, SB, CW), jnp.float32),
            pltpu.VMEM_SHARED((NP, CW), jnp.float32),
            pltpu.VMEM_SHARED((NP, 16), jnp.float32),
        ] + [pltpu.SemaphoreType.DMA] * (2 * G + 1),
    )(h4, asdup, addup, srcr, dstr, z64, z16)


def _stage_c_body(num_ref, den_ref, b1_ref, w2_ref, aa2_ref, exp8_ref,
                  h2_ref, aaout_ref):
    hcat = jnp.concatenate([num_ref[i] for i in range(NCHUNK)], axis=-1)
    den = den_ref[0][:, 0:HEADS]
    recip = 1.0 / (den + 1e-16)
    denex = jnp.dot(recip, exp8_ref[...], preferred_element_type=jnp.float32)
    out1 = hcat * denex + b1_ref[...]
    t = jnp.where(out1 > 0.0, out1, jnp.exp(out1) - 1.0)
    h2 = jnp.dot(t, w2_ref[...], preferred_element_type=jnp.float32)
    h2_ref[...] = h2
    aaout_ref[...] = jnp.dot(h2, aa2_ref[...],
                             preferred_element_type=jnp.float32)


def _stage_c(num1, den1, b1, W2, AA2, EXP8):
    return pl.pallas_call(
        _stage_c_body,
        grid=(NBLK,),
        in_specs=[
            pl.BlockSpec((NCHUNK, BN, CW), lambda i: (0, i, 0)),
            pl.BlockSpec((1, BN, 16), lambda i: (0, i, 0)),
            pl.BlockSpec((1, F1), lambda i: (0, 0)),
            pl.BlockSpec((F1, D_OUT), lambda i: (0, 0)),
            pl.BlockSpec((D_OUT, 32), lambda i: (0, 0)),
            pl.BlockSpec((HEADS, F1), lambda i: (0, 0)),
        ],
        out_specs=[
            pl.BlockSpec((BN, D_OUT), lambda i: (i, 0)),
            pl.BlockSpec((BN, 32), lambda i: (i, 0)),
        ],
        out_shape=[
            jax.ShapeDtypeStruct((N, D_OUT), jnp.float32),
            jax.ShapeDtypeStruct((N, 32), jnp.float32),
        ],
    )(num1, den1, b1, W2, AA2, EXP8)


def _stage_d_body(h2, asd, add_, srcr, dstr, z64, z16,
                  num2, den2,
                  idx_s, idx_d, abufs, bbufs, expbs, hrows,
                  num2_acc, den2_acc, *sems):
    semw = sems[2 * G]
    cid = lax.axis_index("c")
    sid = lax.axis_index("s")
    wid = sid * NC + cid
    rows0 = sid * RPT

    pltpu.sync_copy(srcr.at[wid], idx_s)
    pltpu.sync_copy(dstr.at[wid], idx_d)
    pltpu.sync_copy(z64, num2_acc.at[pl.ds(rows0, RPT)])
    pltpu.sync_copy(z16, den2_acc.at[pl.ds(rows0, RPT)])
    plsc.subcore_barrier()

    def group_body(g, carry):
        b0 = g * G
        wprev = [None]

        def wissue(desc_fn):
            if wprev[0] is not None:
                wprev[0].wait()
            wprev[0] = desc_fn()

        adescs, hdescs = [], []
        for q in range(G):
            b = b0 + q
            adescs.append((
                pltpu.async_copy(asd.at[idx_s.at[b]], abufs.at[q],
                                 sems[q]),
                pltpu.async_copy(add_.at[idx_d.at[b]], bbufs.at[q],
                                 sems[q]),
            ))
            hdescs.append(pltpu.async_copy(
                h2.at[idx_s.at[b]], hrows.at[q], sems[G + q]))
        for q in range(G):
            b = b0 + q
            for d in adescs[q]:
                d.wait()
            _exp_loop(abufs.at[q], bbufs.at[q], expbs.at[q])
            wissue(lambda q=q, b=b: pltpu.async_copy(
                expbs.at[q], den2_acc.at[idx_d.at[b]], semw, add=True))
            hdescs[q].wait()
            _fused_weight_loop(expbs.at[q], hrows.at[q],
                               jnp.zeros((16,), jnp.int32))
            wissue(lambda q=q, b=b: pltpu.async_copy(
                hrows.at[q], num2_acc.at[idx_d.at[b]], semw, add=True))
        wprev[0].wait()
        return carry

    lax.fori_loop(0, NB2 // G, group_body, 0)
    plsc.subcore_barrier()
    pltpu.sync_copy(num2_acc.at[pl.ds(rows0, RPT)],
                    num2.at[cid].at[pl.ds(rows0, RPT)])
    pltpu.sync_copy(den2_acc.at[pl.ds(rows0, RPT)],
                    den2.at[cid].at[pl.ds(rows0, RPT)])


def _stage_d(h2, as2dup, ad2dup, srcr, dstr, z64, z16):
    return pl.kernel(
        _stage_d_body,
        out_type=[
            jax.ShapeDtypeStruct((NC, NP, D_OUT), jnp.float32),
            jax.ShapeDtypeStruct((NC, NP, 16), jnp.float32),
        ],
        mesh=plsc.VectorSubcoreMesh(core_axis_name="c", subcore_axis_name="s"),
        compiler_params=_SC_PARAMS,
        scratch_types=[
            pltpu.VMEM((NB2, SB), jnp.int32),
            pltpu.VMEM((NB2, SB), jnp.int32),
            pltpu.VMEM((G, SB, 16), jnp.float32),
            pltpu.VMEM((G, SB, 16), jnp.float32),
            pltpu.VMEM((G, SB, 16), jnp.float32),
            pltpu.VMEM((G, SB, D_OUT), jnp.float32),
            pltpu.VMEM_SHARED((NP, D_OUT), jnp.float32),
            pltpu.VMEM_SHARED((NP, 16), jnp.float32),
        ] + [pltpu.SemaphoreType.DMA] * (2 * G + 1),
    )(h2, as2dup, ad2dup, srcr, dstr, z64, z16)


def _stage_e_body(num2_ref, den2_ref, b2_ref, out_ref):
    s = num2_ref[0] + num2_ref[1]
    d = den2_ref[0][:, 0:1] + den2_ref[1][:, 0:1]
    out_ref[...] = s / (d + 1e-16) + b2_ref[...]


def _stage_e(num2, den2, b2):
    return pl.pallas_call(
        _stage_e_body,
        grid=(NBLK,),
        in_specs=[
            pl.BlockSpec((NC, BN, D_OUT), lambda i: (0, i, 0)),
            pl.BlockSpec((NC, BN, 16), lambda i: (0, i, 0)),
            pl.BlockSpec((1, D_OUT), lambda i: (0, 0)),
        ],
        out_specs=pl.BlockSpec((BN, D_OUT), lambda i: (i, 0)),
        out_shape=jax.ShapeDtypeStruct((N, D_OUT), jnp.float32),
    )(num2, den2, b2)


def kernel(x, edge_index, W1, a_src1, a_dst1, b1, W2, a_src2, a_dst2, b2):
    src = edge_index[0]
    dst = edge_index[1]

    eye8 = jnp.eye(HEADS, dtype=jnp.float32)
    blk_s = (a_src1[:, :, None] * eye8[:, None, :]).reshape(F1, HEADS)
    blk_d = (a_dst1[:, :, None] * eye8[:, None, :]).reshape(F1, HEADS)
    AA1 = jnp.concatenate([blk_s, blk_s, blk_d, blk_d], axis=1)
    AA1r = AA1.reshape(NCHUNK, CW, 32)
    W1r = W1.reshape(D_IN, NCHUNK, CW).transpose(1, 0, 2)

    AA2 = jnp.concatenate([
        jnp.broadcast_to(a_src2.reshape(D_OUT, 1), (D_OUT, 16)),
        jnp.broadcast_to(a_dst2.reshape(D_OUT, 1), (D_OUT, 16)),
    ], axis=1)
    EXP8 = (eye8[:, :, None] * jnp.ones((1, 1, HID), jnp.float32)
            ).reshape(HEADS, F1)

    h4, aa = _stage_a(x, W1r, AA1r)
    asdup = aa[:, :16]
    addup = aa[:, 16:]

    srcr1 = src.reshape(NS, NB1, SB)
    dstr1 = dst.reshape(NS, NB1, SB)
    z64 = jnp.zeros((RPT, CW), jnp.float32)
    z8 = jnp.zeros((RPT, 16), jnp.float32)

    num1, den1, _ = _stage_b(h4, asdup, addup, srcr1, dstr1, z64, z8)
    h2, aa2o = _stage_c(num1, den1, b1.reshape(1, F1), W2, AA2, EXP8)

    srcr2 = src.reshape(NC * NS, NB2, SB)
    dstr2 = dst.reshape(NC * NS, NB2, SB)
    num2, den2 = _stage_d(h2, aa2o[:, :16], aa2o[:, 16:], srcr2, dstr2,
                          z64, z8)
    return _stage_e(num2, den2, b2.reshape(1, D_OUT))
